# baseline (speedup 1.0000x reference)
import functools

import jax
import jax.numpy as jnp
from jax import lax
from jax.experimental import pallas as pl
from jax.experimental.pallas import tpu as pltpu
from jax.experimental.pallas import tpu_sc as plsc

U = 20000
NI = 30000
N = 50000
E = 800000
H = 32
NLAYER = 3

NC, NS, L = 2, 16, 16
CB = 128
EPAD = 819200
NCHUNK = EPAD // CB // NS
SUP = 25
NSUP = NCHUNK // SUP
RB = 200
NRCH = N // RB
KMAX = -(-NRCH // NS)

_mesh = plsc.VectorSubcoreMesh(
    core_axis_name="c", subcore_axis_name="s", num_cores=NC, num_subcores=NS)


_PB = 2000


def _prep_body(u_ref, it_ref, o_ref):
    i = pl.program_id(0)
    x = jnp.where(i < U // _PB, u_ref[...], it_ref[...])
    col = lax.broadcasted_iota(jnp.int32, x.shape, 1)
    y = jnp.where(col == 0, 0.0, x)
    ysq = jnp.sum(y * y, axis=1, keepdims=True)
    first = jnp.sqrt(jnp.clip(1.0 + ysq, 1e-7, None))
    ynorm = jnp.maximum(jnp.sqrt(ysq), 1e-15)
    theta = jnp.maximum(first, 1.0 + 1e-7)
    arcosh = jnp.log(theta + jnp.sqrt(jnp.clip(theta * theta - 1.0, 1e-15, None)))
    out = y * (arcosh / ynorm)
    o_ref[0] = out[:, :H]
    o_ref[1] = out[:, H:]


def _preprocess(emb_user, emb_item):
    return pl.pallas_call(
        _prep_body,
        grid=((U + NI) // _PB,),
        in_specs=[
            pl.BlockSpec((_PB, 2 * H), lambda i: (jnp.minimum(i, U // _PB - 1), 0)),
            pl.BlockSpec((_PB, 2 * H), lambda i: (jnp.maximum(i - U // _PB, 0), 0)),
        ],
        out_specs=pl.BlockSpec((NC, _PB, H), lambda i: (0, i, 0)),
        out_shape=jax.ShapeDtypeStruct((NC, N, H), jnp.float32),
    )(emb_user, emb_item)


@functools.partial(
    pl.kernel,
    out_type=[
        jax.ShapeDtypeStruct((NC * N, H), jnp.float32)
        for _ in range(NLAYER)
    ],
    mesh=_mesh,
    compiler_params=pltpu.CompilerParams(use_tc_tiling_on_sc=False),
    scratch_types=[
        pltpu.VMEM_SHARED((N, H), jnp.float32),
        pltpu.VMEM((RB, H), jnp.float32),
        pltpu.VMEM((CB, H), jnp.float32),
        pltpu.VMEM((CB, H), jnp.float32),
        pltpu.VMEM((SUP, CB), jnp.int32),
        pltpu.VMEM((SUP, CB), jnp.int32),
        pltpu.VMEM((SUP, CB), jnp.int32),
        pltpu.VMEM((SUP + 1, CB), jnp.float32),
        pltpu.SemaphoreType.DMA,
        pltpu.SemaphoreType.DMA,
    ],
)
def _propagate(all2, src_hbm, dst_hbm, w_hbm, cur1_hbm, cur2_hbm, cur3_hbm,
               accum, tmp, rows_a, rows_b, srcv, gidxv, dstv, wv, sem_a, sem_b):
    c = lax.axis_index("c")
    s = lax.axis_index("s")
    half_base = c * N
    cbase = s * NCHUNK

    zeros = jnp.zeros((L,), jnp.float32)

    def _zero_tmp(t, _):
        tmp[t // 2, pl.ds((t % 2) * L, L)] = zeros
        return 0

    def _edge_loop(table_ref):
        def super_body(u, _):
            c0 = cbase + u * SUP
            pltpu.sync_copy(src_hbm.at[pl.ds(c0, SUP)], srcv)
            pltpu.sync_copy(dst_hbm.at[pl.ds(c0, SUP)], dstv)
            pltpu.sync_copy(w_hbm.at[pl.ds(c0, SUP)], wv.at[pl.ds(0, SUP)])

            def mkidx(r, _):
                for g in range(CB // L):
                    gidxv[r, pl.ds(g * L, L)] = srcv[r, pl.ds(g * L, L)] + half_base
                return 0
            lax.fori_loop(0, SUP, mkidx, 0)

            def issue(j, rows, sem):
                pltpu.async_copy(table_ref.at[gidxv.at[j]], rows, sem)

            def finish(j, rows, sem):
                pltpu.make_async_copy(table_ref.at[gidxv.at[j]], rows, sem).wait()

                def wmul16(g, _):
                    wg = wv[j, pl.ds(g * L, L)]
                    for t in range(L):
                        e = g * L + t
                        rows[e, pl.ds(0, L)] = rows[e, pl.ds(0, L)] * wg[t]
                        rows[e, pl.ds(L, L)] = rows[e, pl.ds(L, L)] * wg[t]
                    return 0
                lax.fori_loop(0, CB // L, wmul16, 0)
                pltpu.sync_copy(rows, accum.at[dstv.at[j]], add=True)

            issue(0, rows_a, sem_a)

            def pair(g, _):
                issue(2 * g + 1, rows_b, sem_b)
                finish(2 * g, rows_a, sem_a)
                issue(2 * g + 2, rows_a, sem_a)
                finish(2 * g + 1, rows_b, sem_b)
                return 0
            lax.fori_loop(0, (SUP - 1) // 2, pair, 0)
            finish(SUP - 1, rows_a, sem_a)
            return 0
        lax.fori_loop(0, NSUP, super_body, 0)

    tables = [all2, cur1_hbm, cur2_hbm, cur3_hbm]
    for layer in range(NLAYER):
        lax.fori_loop(0, 2 * RB, _zero_tmp, 0)
        for k in range(KMAX):
            q = s + k * NS
            @pl.when(q < NRCH)
            def _():
                pltpu.sync_copy(tmp, accum.at[pl.ds(q * RB, RB)])
        plsc.subcore_barrier()

        _edge_loop(tables[layer])
        plsc.subcore_barrier()

        out_hbm = tables[layer + 1]
        for k in range(KMAX):
            q = s + k * NS
            @pl.when(q < NRCH)
            def _():
                pltpu.sync_copy(accum.at[pl.ds(q * RB, RB)], tmp)
                pltpu.sync_copy(tmp, out_hbm.at[pl.ds(half_base + q * RB, RB)])


_GB = 1024 // (NC * NS)


@functools.partial(
    pl.kernel,
    out_type=[
        jax.ShapeDtypeStruct((NC, 1024, H), jnp.float32),
        jax.ShapeDtypeStruct((NC, 1024, H), jnp.float32),
    ],
    mesh=_mesh,
    compiler_params=pltpu.CompilerParams(use_tc_tiling_on_sc=False),
    scratch_types=[
        pltpu.VMEM((_GB,), jnp.int32),
        pltpu.VMEM((_GB,), jnp.int32),
        pltpu.VMEM((_GB, H), jnp.float32),
        pltpu.VMEM((_GB, H), jnp.float32),
        pltpu.SemaphoreType.DMA,
    ],
)
def _pick(cur1_hbm, cur2_hbm, cur3_hbm, user_hbm, pos_hbm, uo_hbm, po_hbm,
          idxv, gv, rbuf, racc, sem):
    c = lax.axis_index("c")
    s = lax.axis_index("s")
    b0 = (s * NC + c) * _GB

    def gather_half(src_idx_hbm, off, out_hbm, h):
        pltpu.sync_copy(src_idx_hbm.at[pl.ds(b0, _GB)], idxv)

        def mk(j, _):
            gv[pl.ds(j * L, L)] = idxv[pl.ds(j * L, L)] + off
            return 0
        lax.fori_loop(0, _GB // L, mk, 0)
        pltpu.async_copy(cur1_hbm.at[gv], racc, sem).wait()
        for tab in (cur2_hbm, cur3_hbm):
            pltpu.async_copy(tab.at[gv], rbuf, sem).wait()

            def acc(t, _):
                i, off2 = t // 2, (t % 2) * L
                racc[i, pl.ds(off2, L)] = (
                    racc[i, pl.ds(off2, L)] + rbuf[i, pl.ds(off2, L)])
                return 0
            lax.fori_loop(0, 2 * _GB, acc, 0)
        pltpu.sync_copy(racc, out_hbm.at[h, pl.ds(b0, _GB)])

    for h in range(NC):
        gather_half(user_hbm, h * N, uo_hbm, h)
        gather_half(pos_hbm, h * N + U, po_hbm, h)


_IB = 2000


def _items_body(c1_ref, c2_ref, c3_ref, o_ref):
    o_ref[:, :H] = c1_ref[0] + c2_ref[0] + c3_ref[0]
    o_ref[:, H:] = c1_ref[1] + c2_ref[1] + c3_ref[1]


def _items(c1, c2, c3):
    spec = pl.BlockSpec((NC, _IB, H), lambda i: (0, U // _IB + i, 0))
    return pl.pallas_call(
        _items_body,
        grid=(NI // _IB,),
        in_specs=[spec, spec, spec],
        out_specs=pl.BlockSpec((_IB, 2 * H), lambda i: (i, 0)),
        out_shape=jax.ShapeDtypeStruct((NI, 2 * H), jnp.float32),
    )(c1, c2, c3)


def kernel(embedding_user, embedding_item, edge_weight, edge_index, user, pos):
    all_half = _preprocess(embedding_user, embedding_item)
    all2 = all_half.reshape(NC * N, H)
    pad = EPAD - E
    src = jnp.pad(edge_index[0], (0, pad)).reshape(EPAD // CB, CB)
    dst = jnp.pad(edge_index[1], (0, pad)).reshape(EPAD // CB, CB)
    w2 = jnp.pad(edge_weight, (0, pad)).reshape(EPAD // CB, CB)
    cur1, cur2, cur3 = _propagate(all2, src, dst, w2)
    items = _items(cur1.reshape(NC, N, H), cur2.reshape(NC, N, H),
                   cur3.reshape(NC, N, H))
    uo, po = _pick(cur1, cur2, cur3, user, pos)
    ori_user = jnp.concatenate([uo[0], uo[1]], axis=1)
    ori_item = jnp.concatenate([po[0], po[1]], axis=1)
    return (ori_user, ori_item, items)


# speedup vs baseline: 1.1876x; 1.1876x over previous
import functools

import jax
import jax.numpy as jnp
from jax import lax
from jax.experimental import pallas as pl
from jax.experimental.pallas import tpu as pltpu
from jax.experimental.pallas import tpu_sc as plsc

U = 20000
NI = 30000
N = 50000
E = 800000
H = 32
NLAYER = 3

NC, NS, L = 2, 16, 16
CB = 80
EPAD = E
NCHUNK = EPAD // CB // NS
SUP = 25
NSUP = NCHUNK // SUP
RB = 400
NRCH = N // RB
KMAX = -(-NRCH // NS)

_mesh = plsc.VectorSubcoreMesh(
    core_axis_name="c", subcore_axis_name="s", num_cores=NC, num_subcores=NS)


_PB = 2000


def _prep_body(u_ref, it_ref, o_ref):
    i = pl.program_id(0)
    x = jnp.where(i < U // _PB, u_ref[...], it_ref[...])
    col = lax.broadcasted_iota(jnp.int32, x.shape, 1)
    y = jnp.where(col == 0, 0.0, x)
    ysq = jnp.sum(y * y, axis=1, keepdims=True)
    first = jnp.sqrt(jnp.clip(1.0 + ysq, 1e-7, None))
    ynorm = jnp.maximum(jnp.sqrt(ysq), 1e-15)
    theta = jnp.maximum(first, 1.0 + 1e-7)
    arcosh = jnp.log(theta + jnp.sqrt(jnp.clip(theta * theta - 1.0, 1e-15, None)))
    out = y * (arcosh / ynorm)
    o_ref[0] = out[:, :H]
    o_ref[1] = out[:, H:]


def _preprocess(emb_user, emb_item):
    return pl.pallas_call(
        _prep_body,
        grid=((U + NI) // _PB,),
        in_specs=[
            pl.BlockSpec((_PB, 2 * H), lambda i: (jnp.minimum(i, U // _PB - 1), 0)),
            pl.BlockSpec((_PB, 2 * H), lambda i: (jnp.maximum(i - U // _PB, 0), 0)),
        ],
        out_specs=pl.BlockSpec((NC, _PB, H), lambda i: (0, i, 0)),
        out_shape=jax.ShapeDtypeStruct((NC, N, H), jnp.float32),
    )(emb_user, emb_item)


@functools.partial(
    pl.kernel,
    out_type=[
        jax.ShapeDtypeStruct((NC * N, H), jnp.float32)
        for _ in range(NLAYER)
    ],
    mesh=_mesh,
    compiler_params=pltpu.CompilerParams(use_tc_tiling_on_sc=False),
    scratch_types=[
        pltpu.VMEM_SHARED((N, H), jnp.float32),
        pltpu.VMEM((RB, H), jnp.float32),
        pltpu.VMEM((CB, H), jnp.float32),
        pltpu.VMEM((CB, H), jnp.float32),
        pltpu.VMEM((SUP, CB), jnp.int32),
        pltpu.VMEM((SUP, CB), jnp.int32),
        pltpu.VMEM((SUP, CB), jnp.int32),
        pltpu.VMEM((SUP + 1, CB), jnp.float32),
        pltpu.SemaphoreType.DMA,
        pltpu.SemaphoreType.DMA,
    ],
)
def _propagate(all2, src_hbm, dst_hbm, w_hbm, cur1_hbm, cur2_hbm, cur3_hbm,
               accum, tmp, rows_a, rows_b, srcv, gidxv, dstv, wv, sem_a, sem_b):
    c = lax.axis_index("c")
    s = lax.axis_index("s")
    half_base = c * N
    cbase = s * NCHUNK

    zeros = jnp.zeros((L,), jnp.float32)

    def _zero_tmp(t, _):
        tmp[t // 2, pl.ds((t % 2) * L, L)] = zeros
        return 0

    def _edge_loop(table_ref):
        def super_body(u, _):
            c0 = cbase + u * SUP
            pltpu.sync_copy(src_hbm.at[pl.ds(c0, SUP)], srcv)
            pltpu.sync_copy(dst_hbm.at[pl.ds(c0, SUP)], dstv)
            pltpu.sync_copy(w_hbm.at[pl.ds(c0, SUP)], wv.at[pl.ds(0, SUP)])

            def mkidx(r, _):
                for g in range(CB // L):
                    gidxv[r, pl.ds(g * L, L)] = srcv[r, pl.ds(g * L, L)] + half_base
                return 0
            lax.fori_loop(0, SUP, mkidx, 0)

            def issue(j, rows, sem):
                pltpu.async_copy(table_ref.at[gidxv.at[j]], rows, sem)

            def finish(j, rows, sem):
                pltpu.make_async_copy(table_ref.at[gidxv.at[j]], rows, sem).wait()

                def wmul16(g, _):
                    wg = wv[j, pl.ds(g * L, L)]
                    for t in range(L):
                        e = g * L + t
                        rows[e, pl.ds(0, L)] = rows[e, pl.ds(0, L)] * wg[t]
                        rows[e, pl.ds(L, L)] = rows[e, pl.ds(L, L)] * wg[t]
                    return 0
                lax.fori_loop(0, CB // L, wmul16, 0)
                pltpu.sync_copy(rows, accum.at[dstv.at[j]], add=True)

            issue(0, rows_a, sem_a)

            def pair(g, _):
                issue(2 * g + 1, rows_b, sem_b)
                finish(2 * g, rows_a, sem_a)
                issue(2 * g + 2, rows_a, sem_a)
                finish(2 * g + 1, rows_b, sem_b)
                return 0
            lax.fori_loop(0, (SUP - 1) // 2, pair, 0)
            finish(SUP - 1, rows_a, sem_a)
            return 0
        lax.fori_loop(0, NSUP, super_body, 0)

    tables = [all2, cur1_hbm, cur2_hbm, cur3_hbm]
    for layer in range(NLAYER):
        lax.fori_loop(0, 2 * RB, _zero_tmp, 0)
        for k in range(KMAX):
            q = s + k * NS
            @pl.when(q < NRCH)
            def _():
                pltpu.sync_copy(tmp, accum.at[pl.ds(q * RB, RB)])
        plsc.subcore_barrier()

        _edge_loop(tables[layer])
        plsc.subcore_barrier()

        out_hbm = tables[layer + 1]
        for k in range(KMAX):
            q = s + k * NS
            @pl.when(q < NRCH)
            def _():
                pltpu.sync_copy(accum.at[pl.ds(q * RB, RB)], tmp)
                pltpu.sync_copy(tmp, out_hbm.at[pl.ds(half_base + q * RB, RB)])


_GB = 1024 // (NC * NS)


@functools.partial(
    pl.kernel,
    out_type=[
        jax.ShapeDtypeStruct((NC, 1024, H), jnp.float32),
        jax.ShapeDtypeStruct((NC, 1024, H), jnp.float32),
    ],
    mesh=_mesh,
    compiler_params=pltpu.CompilerParams(use_tc_tiling_on_sc=False),
    scratch_types=[
        pltpu.VMEM((_GB,), jnp.int32),
        pltpu.VMEM((_GB,), jnp.int32),
        pltpu.VMEM((_GB, H), jnp.float32),
        pltpu.VMEM((_GB, H), jnp.float32),
        pltpu.SemaphoreType.DMA,
    ],
)
def _pick(cur1_hbm, cur2_hbm, cur3_hbm, user_hbm, pos_hbm, uo_hbm, po_hbm,
          idxv, gv, rbuf, racc, sem):
    c = lax.axis_index("c")
    s = lax.axis_index("s")
    b0 = (s * NC + c) * _GB

    def gather_half(src_idx_hbm, off, out_hbm, h):
        pltpu.sync_copy(src_idx_hbm.at[pl.ds(b0, _GB)], idxv)

        def mk(j, _):
            gv[pl.ds(j * L, L)] = idxv[pl.ds(j * L, L)] + off
            return 0
        lax.fori_loop(0, _GB // L, mk, 0)
        pltpu.async_copy(cur1_hbm.at[gv], racc, sem).wait()
        for tab in (cur2_hbm, cur3_hbm):
            pltpu.async_copy(tab.at[gv], rbuf, sem).wait()

            def acc(t, _):
                i, off2 = t // 2, (t % 2) * L
                racc[i, pl.ds(off2, L)] = (
                    racc[i, pl.ds(off2, L)] + rbuf[i, pl.ds(off2, L)])
                return 0
            lax.fori_loop(0, 2 * _GB, acc, 0)
        pltpu.sync_copy(racc, out_hbm.at[h, pl.ds(b0, _GB)])

    for h in range(NC):
        gather_half(user_hbm, h * N, uo_hbm, h)
        gather_half(pos_hbm, h * N + U, po_hbm, h)


_IB = 2000


def _items_body(c1_ref, c2_ref, c3_ref, o_ref):
    o_ref[:, :H] = c1_ref[0] + c2_ref[0] + c3_ref[0]
    o_ref[:, H:] = c1_ref[1] + c2_ref[1] + c3_ref[1]


def _items(c1, c2, c3):
    spec = pl.BlockSpec((NC, _IB, H), lambda i: (0, U // _IB + i, 0))
    return pl.pallas_call(
        _items_body,
        grid=(NI // _IB,),
        in_specs=[spec, spec, spec],
        out_specs=pl.BlockSpec((_IB, 2 * H), lambda i: (i, 0)),
        out_shape=jax.ShapeDtypeStruct((NI, 2 * H), jnp.float32),
    )(c1, c2, c3)


def kernel(embedding_user, embedding_item, edge_weight, edge_index, user, pos):
    all_half = _preprocess(embedding_user, embedding_item)
    all2 = all_half.reshape(NC * N, H)
    src = edge_index[0].reshape(EPAD // CB, CB)
    dst = edge_index[1].reshape(EPAD // CB, CB)
    w2 = edge_weight.reshape(EPAD // CB, CB)
    cur1, cur2, cur3 = _propagate(all2, src, dst, w2)
    items = _items(cur1.reshape(NC, N, H), cur2.reshape(NC, N, H),
                   cur3.reshape(NC, N, H))
    uo, po = _pick(cur1, cur2, cur3, user, pos)
    ori_user = jnp.concatenate([uo[0], uo[1]], axis=1)
    ori_item = jnp.concatenate([po[0], po[1]], axis=1)
    return (ori_user, ori_item, items)


# speedup vs baseline: 1.4642x; 1.2329x over previous
import functools

import jax
import jax.numpy as jnp
from jax import lax
from jax.experimental import pallas as pl
from jax.experimental.pallas import tpu as pltpu
from jax.experimental.pallas import tpu_sc as plsc

U = 20000
NI = 30000
N = 50000
E = 800000
H = 32
NLAYER = 3

NC, NS, L = 2, 16, 16
CB = 80
EPAD = E
NCHUNK = EPAD // CB // NS
SUP = 25
NSUP = NCHUNK // SUP
RB = 200
NRCH = N // RB
KMAX = -(-NRCH // NS)

_mesh = plsc.VectorSubcoreMesh(
    core_axis_name="c", subcore_axis_name="s", num_cores=NC, num_subcores=NS)


_PB = 2000


def _prep_body(u_ref, it_ref, o_ref):
    i = pl.program_id(0)
    x = jnp.where(i < U // _PB, u_ref[...], it_ref[...])
    col = lax.broadcasted_iota(jnp.int32, x.shape, 1)
    y = jnp.where(col == 0, 0.0, x)
    ysq = jnp.sum(y * y, axis=1, keepdims=True)
    first = jnp.sqrt(jnp.clip(1.0 + ysq, 1e-7, None))
    ynorm = jnp.maximum(jnp.sqrt(ysq), 1e-15)
    theta = jnp.maximum(first, 1.0 + 1e-7)
    arcosh = jnp.log(theta + jnp.sqrt(jnp.clip(theta * theta - 1.0, 1e-15, None)))
    out = y * (arcosh / ynorm)
    o_ref[0] = out[:, :H]
    o_ref[1] = out[:, H:]


def _preprocess(emb_user, emb_item):
    return pl.pallas_call(
        _prep_body,
        grid=((U + NI) // _PB,),
        in_specs=[
            pl.BlockSpec((_PB, 2 * H), lambda i: (jnp.minimum(i, U // _PB - 1), 0)),
            pl.BlockSpec((_PB, 2 * H), lambda i: (jnp.maximum(i - U // _PB, 0), 0)),
        ],
        out_specs=pl.BlockSpec((NC, _PB, H), lambda i: (0, i, 0)),
        out_shape=jax.ShapeDtypeStruct((NC, N, H), jnp.float32),
    )(emb_user, emb_item)


@functools.partial(
    pl.kernel,
    out_type=[
        jax.ShapeDtypeStruct((NC * N, H), jnp.float32)
        for _ in range(NLAYER)
    ],
    mesh=_mesh,
    compiler_params=pltpu.CompilerParams(use_tc_tiling_on_sc=False),
    scratch_types=[
        pltpu.VMEM_SHARED((N, H), jnp.float32),
        pltpu.VMEM((RB, H), jnp.float32),
        pltpu.VMEM((CB, H), jnp.float32),
        pltpu.VMEM((CB, H), jnp.float32),
        pltpu.VMEM((CB, H), jnp.float32),
        pltpu.VMEM((CB, H), jnp.float32),
        pltpu.VMEM((SUP, CB), jnp.int32),
        pltpu.VMEM((SUP, CB), jnp.int32),
        pltpu.VMEM((SUP, CB), jnp.int32),
        pltpu.VMEM((SUP + 1, CB), jnp.float32),
        pltpu.SemaphoreType.DMA,
        pltpu.SemaphoreType.DMA,
        pltpu.SemaphoreType.DMA,
        pltpu.SemaphoreType.DMA,
    ],
)
def _propagate(all2, src_hbm, dst_hbm, w_hbm, cur1_hbm, cur2_hbm, cur3_hbm,
               accum, tmp, rows_a, rows_b, rows_c, rows_d, srcv, gidxv, dstv, wv,
               sem_a, sem_b, sem_c, sem_d):
    c = lax.axis_index("c")
    s = lax.axis_index("s")
    half_base = c * N
    cbase = s * NCHUNK

    zeros = jnp.zeros((L,), jnp.float32)

    def _zero_tmp(t, _):
        tmp[t // 2, pl.ds((t % 2) * L, L)] = zeros
        return 0

    def _edge_loop(table_ref):
        def super_body(u, _):
            c0 = cbase + u * SUP
            pltpu.sync_copy(src_hbm.at[pl.ds(c0, SUP)], srcv)
            pltpu.sync_copy(dst_hbm.at[pl.ds(c0, SUP)], dstv)
            pltpu.sync_copy(w_hbm.at[pl.ds(c0, SUP)], wv.at[pl.ds(0, SUP)])

            def mkidx(r, _):
                for g in range(CB // L):
                    gidxv[r, pl.ds(g * L, L)] = srcv[r, pl.ds(g * L, L)] + half_base
                return 0
            lax.fori_loop(0, SUP, mkidx, 0)

            def issue(j, rows, sem):
                pltpu.async_copy(table_ref.at[gidxv.at[j]], rows, sem)

            def finish(j, rows, sem):
                pltpu.make_async_copy(table_ref.at[gidxv.at[j]], rows, sem).wait()

                def wmul16(g, _):
                    wg = wv[j, pl.ds(g * L, L)]
                    for t in range(L):
                        e = g * L + t
                        rows[e, pl.ds(0, L)] = rows[e, pl.ds(0, L)] * wg[t]
                        rows[e, pl.ds(L, L)] = rows[e, pl.ds(L, L)] * wg[t]
                    return 0
                lax.fori_loop(0, CB // L, wmul16, 0)
                pltpu.sync_copy(rows, accum.at[dstv.at[j]], add=True)

            bufs = ((rows_a, sem_a), (rows_b, sem_b), (rows_c, sem_c),
                    (rows_d, sem_d))
            for b in range(3):
                issue(b, *bufs[b])

            def quad(g, _):
                for b in range(4):
                    ch = 4 * g + b
                    finish(ch, *bufs[b])

                    @pl.when(ch + 3 < SUP)
                    def _():
                        issue(ch + 3, *bufs[(b + 3) % 4])
                return 0
            lax.fori_loop(0, SUP // 4, quad, 0)
            finish(SUP - 1, *bufs[(SUP - 1) % 4])
            return 0
        lax.fori_loop(0, NSUP, super_body, 0)

    tables = [all2, cur1_hbm, cur2_hbm, cur3_hbm]
    for layer in range(NLAYER):
        lax.fori_loop(0, 2 * RB, _zero_tmp, 0)
        for k in range(KMAX):
            q = s + k * NS
            @pl.when(q < NRCH)
            def _():
                pltpu.sync_copy(tmp, accum.at[pl.ds(q * RB, RB)])
        plsc.subcore_barrier()

        _edge_loop(tables[layer])
        plsc.subcore_barrier()

        out_hbm = tables[layer + 1]
        for k in range(KMAX):
            q = s + k * NS
            @pl.when(q < NRCH)
            def _():
                pltpu.sync_copy(accum.at[pl.ds(q * RB, RB)], tmp)
                pltpu.sync_copy(tmp, out_hbm.at[pl.ds(half_base + q * RB, RB)])


_GB = 1024 // (NC * NS)


@functools.partial(
    pl.kernel,
    out_type=[
        jax.ShapeDtypeStruct((NC, 1024, H), jnp.float32),
        jax.ShapeDtypeStruct((NC, 1024, H), jnp.float32),
    ],
    mesh=_mesh,
    compiler_params=pltpu.CompilerParams(use_tc_tiling_on_sc=False),
    scratch_types=[
        pltpu.VMEM((_GB,), jnp.int32),
        pltpu.VMEM((_GB,), jnp.int32),
        pltpu.VMEM((_GB, H), jnp.float32),
        pltpu.VMEM((_GB, H), jnp.float32),
        pltpu.SemaphoreType.DMA,
    ],
)
def _pick(cur1_hbm, cur2_hbm, cur3_hbm, user_hbm, pos_hbm, uo_hbm, po_hbm,
          idxv, gv, rbuf, racc, sem):
    c = lax.axis_index("c")
    s = lax.axis_index("s")
    b0 = (s * NC + c) * _GB

    def gather_half(src_idx_hbm, off, out_hbm, h):
        pltpu.sync_copy(src_idx_hbm.at[pl.ds(b0, _GB)], idxv)

        def mk(j, _):
            gv[pl.ds(j * L, L)] = idxv[pl.ds(j * L, L)] + off
            return 0
        lax.fori_loop(0, _GB // L, mk, 0)
        pltpu.async_copy(cur1_hbm.at[gv], racc, sem).wait()
        for tab in (cur2_hbm, cur3_hbm):
            pltpu.async_copy(tab.at[gv], rbuf, sem).wait()

            def acc(t, _):
                i, off2 = t // 2, (t % 2) * L
                racc[i, pl.ds(off2, L)] = (
                    racc[i, pl.ds(off2, L)] + rbuf[i, pl.ds(off2, L)])
                return 0
            lax.fori_loop(0, 2 * _GB, acc, 0)
        pltpu.sync_copy(racc, out_hbm.at[h, pl.ds(b0, _GB)])

    for h in range(NC):
        gather_half(user_hbm, h * N, uo_hbm, h)
        gather_half(pos_hbm, h * N + U, po_hbm, h)


_IB = 2000


def _items_body(c1_ref, c2_ref, c3_ref, o_ref):
    o_ref[:, :H] = c1_ref[0] + c2_ref[0] + c3_ref[0]
    o_ref[:, H:] = c1_ref[1] + c2_ref[1] + c3_ref[1]


def _items(c1, c2, c3):
    spec = pl.BlockSpec((NC, _IB, H), lambda i: (0, U // _IB + i, 0))
    return pl.pallas_call(
        _items_body,
        grid=(NI // _IB,),
        in_specs=[spec, spec, spec],
        out_specs=pl.BlockSpec((_IB, 2 * H), lambda i: (i, 0)),
        out_shape=jax.ShapeDtypeStruct((NI, 2 * H), jnp.float32),
    )(c1, c2, c3)


def kernel(embedding_user, embedding_item, edge_weight, edge_index, user, pos):
    all_half = _preprocess(embedding_user, embedding_item)
    all2 = all_half.reshape(NC * N, H)
    src = edge_index[0].reshape(EPAD // CB, CB)
    dst = edge_index[1].reshape(EPAD // CB, CB)
    w2 = edge_weight.reshape(EPAD // CB, CB)
    cur1, cur2, cur3 = _propagate(all2, src, dst, w2)
    items = _items(cur1.reshape(NC, N, H), cur2.reshape(NC, N, H),
                   cur3.reshape(NC, N, H))
    uo, po = _pick(cur1, cur2, cur3, user, pos)
    ori_user = jnp.concatenate([uo[0], uo[1]], axis=1)
    ori_item = jnp.concatenate([po[0], po[1]], axis=1)
    return (ori_user, ori_item, items)
